# baseline (speedup 1.0000x reference)
import functools

import jax
import jax.numpy as jnp
from jax import lax
from jax.experimental import pallas as pl
from jax.experimental.pallas import tpu as pltpu
from jax.experimental.pallas import tpu_sc as plsc

N = 10000
D = 128
T = 51
TEMPERATURE = 1.5

NC, NS = 2, 16
NW = NC * NS
NP = 10240
ROWS_PER_W = NP // NW
TPAD = 51
BLK = 1
NBLOCKS = ROWS_PER_W // BLK
IDX_W = BLK * TPAD
CHUNK = 40
NCHUNKS = NBLOCKS // CHUNK
LANES = 16


def _norm_body(e1, e2, e3, w, tup, o, otup):
    i = pl.program_id(0)
    f = w[0, 0] * e1[...] + w[0, 1] * e2[...] + w[0, 2] * e3[...]
    n = jnp.sqrt(jnp.sum(f * f, axis=1, keepdims=True))
    rows = i * _NBLK + lax.broadcasted_iota(jnp.int32, (_NBLK, 1), 0)
    o[...] = jnp.where(rows < N, f / jnp.maximum(n, 1e-8), 0.0)
    otup[...] = jnp.where(rows < N, tup[...], 0)


_NBLK = 1024


def _normalize(e1, e2, e3, w, tup):
    return pl.pallas_call(
        _norm_body,
        grid=(NP // _NBLK,),
        in_specs=[
            pl.BlockSpec((_NBLK, D), lambda i: (i, 0)),
            pl.BlockSpec((_NBLK, D), lambda i: (i, 0)),
            pl.BlockSpec((_NBLK, D), lambda i: (i, 0)),
            pl.BlockSpec(memory_space=pltpu.SMEM),
            pl.BlockSpec((_NBLK, TPAD), lambda i: (i, 0)),
        ],
        out_specs=[
            pl.BlockSpec((_NBLK, D), lambda i: (i, 0)),
            pl.BlockSpec((_NBLK, TPAD), lambda i: (i, 0)),
        ],
        out_shape=[
            jax.ShapeDtypeStruct((NP, D), jnp.float32),
            jax.ShapeDtypeStruct((NP, TPAD), jnp.int32),
        ],
    )(e1, e2, e3, w, tup)


def _sc_body(nf_hbm, tup_hbm, out_hbm, idx_ch, idx_b, own_v, own_b, rows0,
             rows1, sims_ch, tab_sh, sem0, sem1, semA, semA2, semB, semB2):
    c = lax.axis_index("c")
    s = lax.axis_index("s")
    w = s * NC + c

    @pl.when(s == 0)
    def _():
        pltpu.sync_copy(nf_hbm, tab_sh)

    plsc.subcore_barrier()

    zero16 = jnp.zeros((LANES,), jnp.float32)
    lanes = lax.iota(jnp.int32, LANES)
    perms = [lanes ^ sh for sh in (8, 4, 2, 1)]
    selbits = [((lanes >> k) & 1) == 1 for k in range(4)]

    def sel_tree(vals):
        lvl = vals
        for k in range(4):
            if len(lvl) == 1:
                break
            lvl = [jnp.where(selbits[k], lvl[2 * m + 1], lvl[2 * m])
                   for m in range(len(lvl) // 2)]
        return lvl[0]

    def compute_block(j, rows_v, ownbuf):
        for i in range(BLK):
            own = [ownbuf[j * BLK + i, pl.ds(16 * k, 16)] for k in range(8)]
            svs = []
            for g in range(4):
                accs = []
                for tt in range(LANES):
                    t = g * LANES + tt
                    if t >= TPAD:
                        accs.append(zero16)
                        continue
                    p = [rows_v[i * TPAD + t, pl.ds(16 * k, 16)] * own[k]
                         for k in range(8)]
                    q = [p[0] + p[1], p[2] + p[3], p[4] + p[5], p[6] + p[7]]
                    acc = (q[0] + q[1]) + (q[2] + q[3])
                    for prm in perms:
                        acc = acc + acc.at[prm].get(mode="promise_in_bounds")
                    accs.append(acc)
                svs.append(sel_tree(accs))
            e0 = jnp.where(lanes == 0, 0.0, jnp.exp(svs[0]))
            e1 = jnp.exp(svs[1])
            e2 = jnp.exp(svs[2])
            e3 = jnp.where(lanes < 3, jnp.exp(svs[3]), 0.0)
            den = (e0 + e1) + (e2 + e3)
            for prm in perms:
                den = den + den.at[prm].get(mode="promise_in_bounds")
            out16 = jnp.where(lanes == 0, svs[0], jnp.where(lanes == 1, den, 0.0))
            sims_ch[j * BLK + i, :] = out16

    def stage(ci, idxbuf, ownbuf, semI, semO):
        pltpu.async_copy(
            tup_hbm.at[pl.ds(w * NBLOCKS + ci * CHUNK, CHUNK)], idxbuf, semI)
        pltpu.async_copy(
            tab_sh.at[pl.ds(w * ROWS_PER_W + ci * CHUNK * BLK, CHUNK * BLK)],
            ownbuf, semO)

    def wait_stage(ci, idxbuf, ownbuf, semI, semO):
        pltpu.make_async_copy(
            tup_hbm.at[pl.ds(w * NBLOCKS + ci * CHUNK, CHUNK)], idxbuf, semI
        ).wait()
        pltpu.make_async_copy(
            tab_sh.at[pl.ds(w * ROWS_PER_W + ci * CHUNK * BLK, CHUNK * BLK)],
            ownbuf, semO).wait()

    def run_chunk(ci, idxbuf, ownbuf):
        pltpu.async_copy(tab_sh.at[idxbuf.at[0]], rows0, sem0)

        def pair_body(kk, carry2):
            j0 = 2 * kk
            j1 = j0 + 1
            pltpu.async_copy(tab_sh.at[idxbuf.at[j1]], rows1, sem1)
            pltpu.make_async_copy(tab_sh.at[idxbuf.at[j0]], rows0, sem0).wait()
            compute_block(j0, rows0, ownbuf)

            @pl.when(kk + 1 < CHUNK // 2)
            def _():
                pltpu.async_copy(tab_sh.at[idxbuf.at[j0 + 2]], rows0, sem0)

            pltpu.make_async_copy(tab_sh.at[idxbuf.at[j1]], rows1, sem1).wait()
            compute_block(j1, rows1, ownbuf)
            return carry2

        lax.fori_loop(0, CHUNK // 2, pair_body, 0)
        pltpu.sync_copy(
            sims_ch,
            out_hbm.at[pl.ds(w * ROWS_PER_W + ci * CHUNK * BLK, CHUNK * BLK)])

    stage(0, idx_ch, own_v, semA, semA2)

    def outer_body(kk, carry):
        c0 = 2 * kk
        c1 = c0 + 1
        stage(c1, idx_b, own_b, semB, semB2)
        wait_stage(c0, idx_ch, own_v, semA, semA2)
        run_chunk(c0, idx_ch, own_v)
        stage(c0 + 2, idx_ch, own_v, semA, semA2)
        wait_stage(c1, idx_b, own_b, semB, semB2)
        run_chunk(c1, idx_b, own_b)
        return carry

    lax.fori_loop(0, NCHUNKS // 2 - 1, outer_body, 0)
    wait_stage(NCHUNKS - 2, idx_ch, own_v, semA, semA2)
    stage(NCHUNKS - 1, idx_b, own_b, semB, semB2)
    run_chunk(NCHUNKS - 2, idx_ch, own_v)
    wait_stage(NCHUNKS - 1, idx_b, own_b, semB, semB2)
    run_chunk(NCHUNKS - 1, idx_b, own_b)


def _sc_sims(nf_pad, tup2d):
    mesh = plsc.VectorSubcoreMesh(
        core_axis_name="c", subcore_axis_name="s", num_cores=NC, num_subcores=NS
    )
    fn = functools.partial(
        pl.kernel,
        mesh=mesh,
        out_type=jax.ShapeDtypeStruct((NP, LANES), jnp.float32),
        scratch_types=[
            pltpu.VMEM((CHUNK, IDX_W), jnp.int32),
            pltpu.VMEM((CHUNK, IDX_W), jnp.int32),
            pltpu.VMEM((CHUNK * BLK, D), jnp.float32),
            pltpu.VMEM((CHUNK * BLK, D), jnp.float32),
            pltpu.VMEM((IDX_W, D), jnp.float32),
            pltpu.VMEM((IDX_W, D), jnp.float32),
            pltpu.VMEM((CHUNK * BLK, LANES), jnp.float32),
            pltpu.VMEM_SHARED((NP, D), jnp.float32),
            pltpu.SemaphoreType.DMA,
            pltpu.SemaphoreType.DMA,
            pltpu.SemaphoreType.DMA,
            pltpu.SemaphoreType.DMA,
            pltpu.SemaphoreType.DMA,
            pltpu.SemaphoreType.DMA,
        ],
    )(_sc_body)
    return fn(nf_pad, tup2d)


def _loss_body(nd_ref, o):
    nd = nd_ref[...]
    term = jnp.log(nd[:, 1:2]) - nd[:, 0:1]
    rmask = lax.broadcasted_iota(jnp.int32, (NP, 1), 0) < N
    o[0, 0] = jnp.sum(jnp.where(rmask, term, 0.0)) / N


def _finalize(sims):
    return pl.pallas_call(
        _loss_body,
        out_specs=pl.BlockSpec(memory_space=pltpu.SMEM),
        out_shape=jax.ShapeDtypeStruct((1, 1), jnp.float32),
    )(sims)


def kernel(emb1, emb2, emb3, weight, tuples):
    nf_pad, tup2d = _normalize(emb1, emb2, emb3, weight, tuples.astype(jnp.int32))
    nd = _sc_sims(nf_pad, tup2d)
    loss = _finalize(nd)
    return loss[0, 0]


# speedup vs baseline: 1.0114x; 1.0114x over previous
import functools

import jax
import jax.numpy as jnp
from jax import lax
from jax.experimental import pallas as pl
from jax.experimental.pallas import tpu as pltpu
from jax.experimental.pallas import tpu_sc as plsc

N = 10000
D = 128
T = 51
TEMPERATURE = 1.5

NC, NS = 2, 16
NW = NC * NS
NP = 10240
ROWS_PER_W = NP // NW
TPAD = 51
BLK = 1
NBLOCKS = ROWS_PER_W // BLK
IDX_W = BLK * TPAD
CHUNK = 64
NCHUNKS = NBLOCKS // CHUNK
LANES = 16


def _norm_body(e1, e2, e3, w, tup, o, otup):
    i = pl.program_id(0)
    f = w[0, 0] * e1[...] + w[0, 1] * e2[...] + w[0, 2] * e3[...]
    n = jnp.sqrt(jnp.sum(f * f, axis=1, keepdims=True))
    rows = i * _NBLK + lax.broadcasted_iota(jnp.int32, (_NBLK, 1), 0)
    o[...] = jnp.where(rows < N, f / jnp.maximum(n, 1e-8), 0.0)
    otup[...] = jnp.where(rows < N, tup[...], 0)


_NBLK = 1024


def _normalize(e1, e2, e3, w, tup):
    return pl.pallas_call(
        _norm_body,
        grid=(NP // _NBLK,),
        in_specs=[
            pl.BlockSpec((_NBLK, D), lambda i: (i, 0)),
            pl.BlockSpec((_NBLK, D), lambda i: (i, 0)),
            pl.BlockSpec((_NBLK, D), lambda i: (i, 0)),
            pl.BlockSpec(memory_space=pltpu.SMEM),
            pl.BlockSpec((_NBLK, TPAD), lambda i: (i, 0)),
        ],
        out_specs=[
            pl.BlockSpec((_NBLK, D), lambda i: (i, 0)),
            pl.BlockSpec((_NBLK, TPAD), lambda i: (i, 0)),
        ],
        out_shape=[
            jax.ShapeDtypeStruct((NP, D), jnp.float32),
            jax.ShapeDtypeStruct((NP, TPAD), jnp.int32),
        ],
    )(e1, e2, e3, w, tup)


def _sc_body(nf_hbm, tup_hbm, out_hbm, idx_ch, own_v, rows0, rows1, sims_ch,
             tab_sh, sem0, sem1):
    c = lax.axis_index("c")
    s = lax.axis_index("s")
    w = s * NC + c

    @pl.when(s == 0)
    def _():
        pltpu.sync_copy(nf_hbm, tab_sh)

    plsc.subcore_barrier()

    zero16 = jnp.zeros((LANES,), jnp.float32)
    lanes = lax.iota(jnp.int32, LANES)
    perms = [lanes ^ sh for sh in (8, 4, 2, 1)]
    selbits = [((lanes >> k) & 1) == 1 for k in range(4)]

    def sel_tree(vals):
        lvl = vals
        for k in range(4):
            if len(lvl) == 1:
                break
            lvl = [jnp.where(selbits[k], lvl[2 * m + 1], lvl[2 * m])
                   for m in range(len(lvl) // 2)]
        return lvl[0]

    def compute_block(j, rows_v):
        for i in range(BLK):
            own = [own_v[j * BLK + i, pl.ds(16 * k, 16)] for k in range(8)]
            svs = []
            for g in range(4):
                accs = []
                for tt in range(LANES):
                    t = g * LANES + tt
                    if t >= TPAD:
                        accs.append(zero16)
                        continue
                    p = [rows_v[i * TPAD + t, pl.ds(16 * k, 16)] * own[k]
                         for k in range(8)]
                    q = [p[0] + p[1], p[2] + p[3], p[4] + p[5], p[6] + p[7]]
                    acc = (q[0] + q[1]) + (q[2] + q[3])
                    for prm in perms:
                        acc = acc + acc.at[prm].get(mode="promise_in_bounds")
                    accs.append(acc)
                svs.append(sel_tree(accs))
            e0 = jnp.where(lanes == 0, 0.0, jnp.exp(svs[0]))
            e1 = jnp.exp(svs[1])
            e2 = jnp.exp(svs[2])
            e3 = jnp.where(lanes < 3, jnp.exp(svs[3]), 0.0)
            den = (e0 + e1) + (e2 + e3)
            for prm in perms:
                den = den + den.at[prm].get(mode="promise_in_bounds")
            out16 = jnp.where(lanes == 0, svs[0], jnp.where(lanes == 1, den, 0.0))
            sims_ch[j * BLK + i, :] = out16

    def chunk_body(ch, carry):
        base_blk = ch * CHUNK
        pltpu.sync_copy(tup_hbm.at[pl.ds(w * NBLOCKS + base_blk, CHUNK)], idx_ch)
        pltpu.sync_copy(
            tab_sh.at[pl.ds(w * ROWS_PER_W + base_blk * BLK, CHUNK * BLK)],
            own_v)
        pltpu.async_copy(tab_sh.at[idx_ch.at[0]], rows0, sem0)

        def pair_body(kk, carry2):
            j0 = 2 * kk
            j1 = j0 + 1
            pltpu.async_copy(tab_sh.at[idx_ch.at[j1]], rows1, sem1)
            pltpu.make_async_copy(tab_sh.at[idx_ch.at[j0]], rows0, sem0).wait()
            compute_block(j0, rows0)

            @pl.when(kk + 1 < CHUNK // 2)
            def _():
                pltpu.async_copy(tab_sh.at[idx_ch.at[j0 + 2]], rows0, sem0)

            pltpu.make_async_copy(tab_sh.at[idx_ch.at[j1]], rows1, sem1).wait()
            compute_block(j1, rows1)
            return carry2

        lax.fori_loop(0, CHUNK // 2, pair_body, 0)
        pltpu.sync_copy(
            sims_ch,
            out_hbm.at[pl.ds(w * ROWS_PER_W + base_blk * BLK, CHUNK * BLK)])
        return carry

    lax.fori_loop(0, NCHUNKS, chunk_body, 0)


def _sc_sims(nf_pad, tup2d):
    mesh = plsc.VectorSubcoreMesh(
        core_axis_name="c", subcore_axis_name="s", num_cores=NC, num_subcores=NS
    )
    fn = functools.partial(
        pl.kernel,
        mesh=mesh,
        out_type=jax.ShapeDtypeStruct((NP, LANES), jnp.float32),
        scratch_types=[
            pltpu.VMEM((CHUNK, IDX_W), jnp.int32),
            pltpu.VMEM((CHUNK * BLK, D), jnp.float32),
            pltpu.VMEM((IDX_W, D), jnp.float32),
            pltpu.VMEM((IDX_W, D), jnp.float32),
            pltpu.VMEM((CHUNK * BLK, LANES), jnp.float32),
            pltpu.VMEM_SHARED((NP, D), jnp.float32),
            pltpu.SemaphoreType.DMA,
            pltpu.SemaphoreType.DMA,
        ],
    )(_sc_body)
    return fn(nf_pad, tup2d)


def _loss_body(nd_ref, o):
    nd = nd_ref[...]
    term = jnp.log(nd[:, 1:2]) - nd[:, 0:1]
    rmask = lax.broadcasted_iota(jnp.int32, (NP, 1), 0) < N
    o[0, 0] = jnp.sum(jnp.where(rmask, term, 0.0)) / N


def _finalize(sims):
    return pl.pallas_call(
        _loss_body,
        out_specs=pl.BlockSpec(memory_space=pltpu.SMEM),
        out_shape=jax.ShapeDtypeStruct((1, 1), jnp.float32),
    )(sims)


def kernel(emb1, emb2, emb3, weight, tuples):
    nf_pad, tup2d = _normalize(emb1, emb2, emb3, weight, tuples.astype(jnp.int32))
    nd = _sc_sims(nf_pad, tup2d)
    loss = _finalize(nd)
    return loss[0, 0]
